# baseline (speedup 1.0000x reference)
import functools
import math

import jax
import jax.numpy as jnp
from jax.experimental import pallas as pl

_NG = 32
_H = 512
_R = 256


def _pad_to(n, m):
    return (n + m - 1) // m * m


def _leaky(h):
    return jnp.where(h >= 0, h, 0.01 * h)


def _dot(a, b):
    return jax.lax.dot_general(a, b, (((1,), (0,)), ((), ())),
                               preferred_element_type=jnp.float32)


def _row_spec(f):
    return pl.BlockSpec((_R, f), lambda i: (i, 0))


def _res_spec(shape):
    return pl.BlockSpec(shape, lambda i: (0, 0))


def _blk_rowmask(n_true):
    i = pl.program_id(0)
    ri = i * _R + jax.lax.broadcasted_iota(jnp.int32, (_R, 1), 0)
    return (ri < n_true).astype(jnp.float32)


def _stat_row(s):
    return jnp.concatenate(
        [s.reshape(1, -1), jnp.zeros((7, s.shape[-1]), jnp.float32)], axis=0)


def _acc_stat(ref, s):
    st = _stat_row(s)

    @pl.when(pl.program_id(0) == 0)
    def _init():
        ref[...] = st

    @pl.when(pl.program_id(0) != 0)
    def _acc():
        ref[...] = ref[...] + st


from jax import lax
from jax.experimental.pallas import tpu as pltpu
from jax.experimental.pallas import tpu_sc as plsc

_K = 128
_KT = 32


def _sc_segsum(slices, rowi, coli, zeros, np_):
    nsl = len(slices)
    E = rowi.shape[0]
    npt = E // 16
    nfull = npt // _K
    ntail = (npt - nfull * _K) // _KT
    rpt = np_ // 16
    mesh = plsc.VectorSubcoreMesh(core_axis_name="c", subcore_axis_name="s")

    @functools.partial(
        pl.kernel, mesh=mesh,
        out_type=[jax.ShapeDtypeStruct((np_, 128), jnp.float32)] * nsl,
        scratch_types=[
            pltpu.VMEM_SHARED((np_, 128), jnp.float32),
            pltpu.VMEM((_K,), jnp.int32),
            pltpu.VMEM((_K,), jnp.int32),
            pltpu.VMEM((_K, 128), jnp.float32),
            pltpu.VMEM((_KT,), jnp.int32),
            pltpu.VMEM((_KT,), jnp.int32),
            pltpu.VMEM((_KT, 128), jnp.float32),
            pltpu.SemaphoreType.DMA,
        ],
    )
    def k(*refs):
        ins = refs[:nsl]
        row_ref = refs[nsl]
        col_ref = refs[nsl + 1]
        zero_ref = refs[nsl + 2]
        outs = refs[nsl + 3:nsl + 3 + nsl]
        acc, ridx, cidx, rows, ridx2, cidx2, rows2, sem = refs[nsl + 3 + nsl:]
        c = lax.axis_index("c")
        s = lax.axis_index("s")
        for j in range(nsl):
            owner = j % 2

            @pl.when(c == owner)
            def _(j=j):
                pltpu.sync_copy(zero_ref.at[pl.ds(s * rpt, rpt)],
                                acc.at[pl.ds(s * rpt, rpt)])
                plsc.subcore_barrier()
                base = s * npt

                def chunk(i, carry):
                    eb = base + i * _K
                    pltpu.sync_copy(row_ref.at[pl.ds(eb, _K)], ridx)
                    pltpu.sync_copy(col_ref.at[pl.ds(eb, _K)], cidx)
                    pltpu.async_copy(ins[j].at[ridx], rows, sem).wait()
                    pltpu.sync_copy(rows, acc.at[cidx], add=True)
                    return carry

                lax.fori_loop(0, nfull, chunk, 0)

                def tailchunk(i, carry):
                    eb = base + nfull * _K + i * _KT
                    pltpu.sync_copy(row_ref.at[pl.ds(eb, _KT)], ridx2)
                    pltpu.sync_copy(col_ref.at[pl.ds(eb, _KT)], cidx2)
                    pltpu.async_copy(ins[j].at[ridx2], rows2, sem).wait()
                    pltpu.sync_copy(rows2, acc.at[cidx2], add=True)
                    return carry

                lax.fori_loop(0, ntail, tailchunk, 0)
                plsc.subcore_barrier()
                pltpu.sync_copy(acc.at[pl.ds(s * rpt, rpt)],
                                outs[j].at[pl.ds(s * rpt, rpt)])

    return k(*slices, rowi, coli, zeros)


def _mlpA_body(n_true, first, *refs):
    if first:
        (w_ref, p_ref, eps_ref, w1_ref, b1_ref, w2_ref, b2_ref,
         a2_ref, mask_ref, st_ref) = refs
        h = w_ref[:, 0:1]
        maskp = w_ref[:, 1:2]
        agg = p_ref[:, 0:1]
        aggm = p_ref[:, 1:2]
    else:
        (s0, s1, s2, s3, s4, p0, p1, p2, p3, pm, eps_ref, w1_ref, b1_ref,
         w2_ref, b2_ref, a2_ref, mask_ref, st_ref) = refs
        h = jnp.concatenate([s0[...], s1[...], s2[...], s3[...]], axis=1)
        maskp = s4[:, 0:1]
        agg = jnp.concatenate([p0[...], p1[...], p2[...], p3[...]], axis=1)
        aggm = pm[:, 0:1]
    rm = _blk_rowmask(n_true)
    z = (1.0 + eps_ref[0, 0]) * h + agg
    if first:
        a1 = jnp.maximum(z * w1_ref[0:1, :] + b1_ref[0:1, :], 0.0)
    else:
        a1 = jnp.maximum(_dot(z, w1_ref[...]) + b1_ref[0:1, :], 0.0)
    a2 = jnp.maximum(_dot(a1, w2_ref[...]) + b2_ref[0:1, :], 0.0)
    a2_ref[...] = a2
    mask = jnp.maximum(maskp, (aggm > 0.0).astype(jnp.float32))
    mask_ref[...] = jnp.broadcast_to(mask, mask_ref.shape)
    _acc_stat(st_ref, jnp.sum(a2 * rm, axis=0))


def _var_body(n_true, x_ref, sum_ref, out_ref):
    rm = _blk_rowmask(n_true)
    mean = sum_ref[0:1, :] * (1.0 / float(n_true))
    d = (x_ref[...] - mean) * rm
    _acc_stat(out_ref, jnp.sum(d * d, axis=0))


def _var_call(n, x, st):
    np_ = x.shape[0]
    return pl.pallas_call(
        functools.partial(_var_body, n),
        grid=(np_ // _R,),
        in_specs=[_row_spec(_H), _res_spec((8, _H))],
        out_specs=_res_spec((8, _H)),
        out_shape=jax.ShapeDtypeStruct((8, _H), jnp.float32),
    )(x, st)


def _bn(x, sum_ref, sq_ref, g, b, n_true):
    inv_n = 1.0 / float(n_true)
    mean = sum_ref[0:1, :] * inv_n
    var = sq_ref[0:1, :] * inv_n
    return (x - mean) / jnp.sqrt(var + 1e-5) * g + b


def _mlpB_body(n_true, first, *refs):
    if first:
        (a2_ref, mask_ref, s1_ref, q1_ref, g_ref, b_ref,
         v_ref, st2_ref) = refs
    else:
        (a2_ref, s0, s1, s2, s3, mask_ref, s1_ref, q1_ref, g_ref, b_ref,
         v_ref, st2_ref) = refs
    rm = _blk_rowmask(n_true)
    l = _leaky(_bn(a2_ref[...], s1_ref, q1_ref, g_ref[0:1, :],
                   b_ref[0:1, :], n_true))
    if not first:
        h = jnp.concatenate([s0[...], s1[...], s2[...], s3[...]], axis=1)
        l = h + l
    mask = mask_ref[:, 0:1]
    v = l * mask / math.sqrt(float(n_true))
    v_ref[...] = v
    _acc_stat(st2_ref, jnp.sum(v * rm, axis=0))


def _mlpC_body(n_true, v_ref, mask_ref, s2_ref, q2_ref, g_ref, b_ref,
               o0_ref, o1_ref, o2_ref, o3_ref, o4_ref):
    hn = _bn(v_ref[...], s2_ref, q2_ref, g_ref[0:1, :], b_ref[0:1, :],
             n_true)
    o0_ref[...] = hn[:, 0:128]
    o1_ref[...] = hn[:, 128:256]
    o2_ref[...] = hn[:, 256:384]
    o3_ref[...] = hn[:, 384:512]
    o4_ref[...] = jnp.concatenate(
        [mask_ref[:, 0:1], jnp.zeros((_R, 127), jnp.float32)], axis=1)


def _headA_body(n_true, s0, s1, s2, s3, s4, b_ref, l1w_ref, l1b_ref,
                l2w_ref, l2b_ref, hf_ref, gst_ref):
    h = jnp.concatenate([s0[...], s1[...], s2[...], s3[...]], axis=1)
    mask = s4[:, 0:1]
    a1 = _leaky(_dot(h, l1w_ref[...]) + l1b_ref[0:1, :]) * mask
    a2 = _leaky(_dot(a1, l2w_ref[...]) + l2b_ref[0:1, :])
    hf = a2[:, 0:1] * mask
    hf_ref[...] = jnp.broadcast_to(hf, hf_ref.shape)
    gid = jax.lax.broadcasted_iota(jnp.int32, (1, 128), 1)
    onehot = b_ref[...] == gid
    hb = jnp.broadcast_to(hf, (_R, 128))
    gmax = jnp.max(jnp.where(onehot, hb, -jnp.inf), axis=0, keepdims=True)
    gmin = jnp.min(jnp.where(onehot, hb, jnp.inf), axis=0, keepdims=True)

    @pl.when(pl.program_id(0) == 0)
    def _init():
        gst_ref[...] = jnp.concatenate(
            [gmax, gmin, jnp.zeros((6, 128), jnp.float32)], axis=0)

    @pl.when(pl.program_id(0) != 0)
    def _acc():
        gst_ref[...] = jnp.concatenate(
            [jnp.maximum(gst_ref[0:1, :], gmax),
             jnp.minimum(gst_ref[1:2, :], gmin),
             jnp.zeros((6, 128), jnp.float32)], axis=0)


def _headB_body(hf_ref, b_ref, gst_ref, o_ref):
    hf = hf_ref[:, 0:1]
    gid = jax.lax.broadcasted_iota(jnp.int32, (1, 128), 1)
    onehot = b_ref[...] == gid
    bmax = jnp.sum(jnp.where(onehot, gst_ref[0:1, :], 0.0), axis=1,
                   keepdims=True)
    bmin = jnp.sum(jnp.where(onehot, gst_ref[1:2, :], 0.0), axis=1,
                   keepdims=True)
    o_ref[...] = (hf - bmin) / (bmax + 1e-6 - bmin)


def _gin_layer(n, np_, W, P, p, bn_g, bn_b, first):
    grid = (np_ // _R,)
    eps = p['eps'].reshape(1, 1)
    w1 = p['w1'][0].reshape(1, -1) if first else p['w1']
    b1 = p['b1'].reshape(1, -1)
    b2 = p['b2'].reshape(1, -1)
    cg = p['bn_g'].reshape(1, -1)
    cb = p['bn_b'].reshape(1, -1)
    g2 = bn_g.reshape(1, -1)
    b2n = bn_b.reshape(1, -1)
    ws = (W,) if first else tuple(W)
    ps = (P,) if first else tuple(P)
    nw = len(ws)

    a2, maskb, s1 = pl.pallas_call(
        functools.partial(_mlpA_body, n, first),
        grid=grid,
        in_specs=(
            [_row_spec(128)] * (2 * nw)
            + [_res_spec((1, 1)), _res_spec(w1.shape), _res_spec((1, _H)),
               _res_spec((_H, _H)), _res_spec((1, _H))]),
        out_specs=[_row_spec(_H), _row_spec(128), _res_spec((8, _H))],
        out_shape=[
            jax.ShapeDtypeStruct((np_, _H), jnp.float32),
            jax.ShapeDtypeStruct((np_, 128), jnp.float32),
            jax.ShapeDtypeStruct((8, _H), jnp.float32),
        ],
    )(*ws, *ps, eps, w1, b1, p['w2'], b2)

    q1 = _var_call(n, a2, s1)

    hs = () if first else ws[:4]
    v, s2 = pl.pallas_call(
        functools.partial(_mlpB_body, n, first),
        grid=grid,
        in_specs=(
            [_row_spec(_H)] + [_row_spec(128)] * len(hs) + [_row_spec(128)]
            + [_res_spec((8, _H)), _res_spec((8, _H)),
               _res_spec((1, _H)), _res_spec((1, _H))]),
        out_specs=[_row_spec(_H), _res_spec((8, _H))],
        out_shape=[
            jax.ShapeDtypeStruct((np_, _H), jnp.float32),
            jax.ShapeDtypeStruct((8, _H), jnp.float32),
        ],
    )(a2, *hs, maskb, s1, q1, cg, cb)

    q2 = _var_call(n, v, s2)

    return pl.pallas_call(
        functools.partial(_mlpC_body, n),
        grid=grid,
        in_specs=[
            _row_spec(_H), _row_spec(128), _res_spec((8, _H)),
            _res_spec((8, _H)), _res_spec((1, _H)), _res_spec((1, _H)),
        ],
        out_specs=[_row_spec(128)] * 5,
        out_shape=[jax.ShapeDtypeStruct((np_, 128), jnp.float32)] * 5,
    )(v, maskb, s2, q2, g2, b2n)


def kernel(x, edge_index, batch, params):
    n = x.shape[0]
    np_ = _pad_to(n, _R)
    grid = (np_ // _R,)

    rowi = edge_index[0]
    coli = edge_index[1]

    xpad = jnp.pad(x, (0, np_ - n))[:, None]
    mask0 = (jnp.abs(xpad) > 0.0).astype(jnp.float32)
    w0 = jnp.concatenate(
        [xpad, mask0, jnp.zeros((np_, 126), jnp.float32)], axis=1)
    zeros = jnp.zeros((np_, 128), jnp.float32)

    bpad = jnp.pad(batch.astype(jnp.int32), (0, np_ - n),
                   constant_values=_NG)[:, None]

    (p1,) = _sc_segsum([w0], rowi, coli, zeros, np_)
    slices = _gin_layer(n, np_, w0, p1, params['conv1'],
                        params['bn1_g'], params['bn1_b'], first=True)

    for p, bn in zip(params['convs'], params['bns']):
        pw = _sc_segsum(list(slices), rowi, coli, zeros, np_)
        slices = _gin_layer(n, np_, slices, pw, p, bn['g'], bn['b'],
                            first=False)

    hid = params['lin1_w'].shape[1]
    hidp = _pad_to(hid, 128)
    l1w = jnp.pad(params['lin1_w'], ((0, 0), (0, hidp - hid)))
    l1b = jnp.pad(params['lin1_b'], (0, hidp - hid)).reshape(1, -1)
    l2w = jnp.pad(params['lin2_w'], ((0, hidp - hid), (0, 127)))
    l2b = jnp.pad(params['lin2_b'], (0, 127)).reshape(1, -1)

    hf, gst = pl.pallas_call(
        functools.partial(_headA_body, n),
        grid=grid,
        in_specs=[
            _row_spec(128)] * 5 + [
            pl.BlockSpec((_R, 1), lambda i: (i, 0)),
            _res_spec((_H, hidp)), _res_spec((1, hidp)),
            _res_spec((hidp, 128)), _res_spec((1, 128)),
        ],
        out_specs=[_row_spec(128), _res_spec((8, 128))],
        out_shape=[
            jax.ShapeDtypeStruct((np_, 128), jnp.float32),
            jax.ShapeDtypeStruct((8, 128), jnp.float32),
        ],
    )(*slices, bpad, l1w, l1b, l2w, l2b)

    probs = pl.pallas_call(
        _headB_body,
        grid=grid,
        in_specs=[
            _row_spec(128), pl.BlockSpec((_R, 1), lambda i: (i, 0)),
            _res_spec((8, 128)),
        ],
        out_specs=pl.BlockSpec((_R, 1), lambda i: (i, 0)),
        out_shape=jax.ShapeDtypeStruct((np_, 1), jnp.float32),
    )(hf, bpad, gst)
    return probs[:n]
